# baseline (speedup 1.0000x reference)
import functools
import jax
import jax.numpy as jnp
from jax import lax
from jax.experimental import pallas as pl
from jax.experimental.pallas import tpu as pltpu
from jax.experimental.pallas import tpu_sc as plsc

_NN = 4
_S = 200
_D = 64
_B = 4096
_ROWS = _B * _S

_NC = 2
_NS = 16
_NW = _NC * _NS

_RPW = _ROWS // _NW
_CHUNK = 128
_NCHUNK = _RPW // _CHUNK

_TPAD = 1024
_SPT = _TPAD // (_NN * _NS)
_TROWS = _SPT * _NN


def _pe_matrix():
    i_num = jnp.arange(0.0, _S, dtype=jnp.float32).reshape(-1, 1)
    j_denom = jnp.power(
        10000.0, jnp.arange(0.0, _D, 2.0, dtype=jnp.float32) / _D
    )
    pe = jnp.zeros((_S, _D), dtype=jnp.float32)
    pe = pe.at[:, 0::2].set(jnp.sin(i_num / j_denom))
    pe = pe.at[:, 1::2].set(jnp.cos(i_num / j_denom))
    return pe


def _sc_body(x_hbm, pe_hbm, emb_hbm, out_hbm,
             x_v, idx_v, rows_v, pe_v, emb_v, t_v, t_shared, sem):
    cid = lax.axis_index("c")
    sid = lax.axis_index("s")
    wid = sid * _NC + cid

    pltpu.sync_copy(pe_hbm.at[pl.ds(sid * _SPT * _D, _SPT * _D)], pe_v)
    pltpu.sync_copy(emb_hbm, emb_v)
    for s_local in range(_SPT):
        for v in range(_NN):
            for k in range(_D // 16):
                pe16 = pe_v[pl.ds(s_local * _D + k * 16, 16)]
                e16 = emb_v[v, pl.ds(k * 16, 16)]
                t_v[s_local * _NN + v, pl.ds(k * 16, 16)] = pe16 + e16
    pltpu.sync_copy(t_v, t_shared.at[pl.ds(sid * _TROWS, _TROWS), :])
    plsc.subcore_barrier()

    lane = lax.iota(jnp.int32, 16)

    def chunk_body(i, carry):
        base = wid * _RPW + i * _CHUNK
        pltpu.sync_copy(x_hbm.at[pl.ds(base, _CHUNK)], x_v)
        for j in range(_CHUNK // 16):
            pos = base + j * 16 + lane
            s_pos = lax.rem(pos, _S)
            x16 = x_v[pl.ds(j * 16, 16)]
            idx_v[pl.ds(j * 16, 16)] = s_pos * _NN + x16
        pltpu.async_copy(t_shared.at[idx_v], rows_v, sem).wait()
        pltpu.sync_copy(rows_v, out_hbm.at[pl.ds(base, _CHUNK), :])
        return carry

    lax.fori_loop(0, _NCHUNK, chunk_body, 0)


@jax.jit
def kernel(X, emb):
    X = X.astype(jnp.int32).reshape(-1)
    pe_flat = jnp.concatenate(
        [_pe_matrix().reshape(-1),
         jnp.zeros(_NS * _SPT * _D - _S * _D, dtype=jnp.float32)]
    )
    mesh = plsc.VectorSubcoreMesh(core_axis_name="c", subcore_axis_name="s")
    out = pl.kernel(
        _sc_body,
        mesh=mesh,
        compiler_params=pltpu.CompilerParams(use_tc_tiling_on_sc=False),
        out_type=jax.ShapeDtypeStruct((_ROWS, _D), jnp.float32),
        scratch_types=[
            pltpu.VMEM((_CHUNK,), jnp.int32),
            pltpu.VMEM((_CHUNK,), jnp.int32),
            pltpu.VMEM((_CHUNK, _D), jnp.float32),
            pltpu.VMEM((_SPT * _D,), jnp.float32),
            pltpu.VMEM((_NN, _D), jnp.float32),
            pltpu.VMEM((_TROWS, _D), jnp.float32),
            pltpu.VMEM_SHARED((_TPAD, _D), jnp.float32),
            pltpu.SemaphoreType.DMA,
        ],
    )(X, pe_flat, emb)
    return out.reshape(_B, _S, _D)


# speedup vs baseline: 1.2492x; 1.2492x over previous
import functools
import jax
import jax.numpy as jnp
from jax import lax
from jax.experimental import pallas as pl
from jax.experimental.pallas import tpu as pltpu
from jax.experimental.pallas import tpu_sc as plsc

_NN = 4
_S = 200
_D = 64
_B = 4096
_ROWS = _B * _S

_NC = 2
_NS = 16
_NW = _NC * _NS

_RPW = _ROWS // _NW
_IDXW = 128
_GPC = 4
_CHUNK = _IDXW * _GPC
_NCHUNK = _RPW // _CHUNK

_TPAD = 1024
_SPT = _TPAD // (_NN * _NS)
_TROWS = _SPT * _NN


def _pe_matrix():
    i_num = jnp.arange(0.0, _S, dtype=jnp.float32).reshape(-1, 1)
    j_denom = jnp.power(
        10000.0, jnp.arange(0.0, _D, 2.0, dtype=jnp.float32) / _D
    )
    pe = jnp.zeros((_S, _D), dtype=jnp.float32)
    pe = pe.at[:, 0::2].set(jnp.sin(i_num / j_denom))
    pe = pe.at[:, 1::2].set(jnp.cos(i_num / j_denom))
    return pe


def _sc_body(x_hbm, pe_hbm, emb_hbm, out_hbm,
             x_v0, x_v1, idx_v0, idx_v1, rows_v0, rows_v1,
             pe_v, emb_v, t_v, t_shared,
             sem_g, sem_o0, sem_o1):
    cid = lax.axis_index("c")
    sid = lax.axis_index("s")
    wid = sid * _NC + cid

    pltpu.sync_copy(pe_hbm.at[pl.ds(sid * _SPT * _D, _SPT * _D)], pe_v)
    pltpu.sync_copy(emb_hbm, emb_v)
    for s_local in range(_SPT):
        for v in range(_NN):
            for k in range(_D // 16):
                pe16 = pe_v[pl.ds(s_local * _D + k * 16, 16)]
                e16 = emb_v[v, pl.ds(k * 16, 16)]
                t_v[s_local * _NN + v, pl.ds(k * 16, 16)] = pe16 + e16
    pltpu.sync_copy(t_v, t_shared.at[pl.ds(sid * _TROWS, _TROWS), :])
    plsc.subcore_barrier()

    lane = lax.iota(jnp.int32, 16)

    def do_chunk(c, i, x_v, idx_v, rows_v, sem_o):
        base = wid * _RPW + c * _CHUNK
        pltpu.sync_copy(x_hbm.at[pl.ds(base, _CHUNK)], x_v)
        for j in range(_CHUNK // 16):
            pos = base + j * 16 + lane
            s_pos = lax.rem(pos, _S)
            x16 = x_v[pl.ds(j * 16, 16)]
            idx_v[j // 8, pl.ds((j % 8) * 16, 16)] = s_pos * _NN + x16
        @pl.when(i > 0)
        def _():
            pltpu.make_async_copy(
                rows_v, out_hbm.at[pl.ds(0, _CHUNK), :], sem_o
            ).wait()
        handles = [
            pltpu.async_copy(
                t_shared.at[idx_v.at[g]],
                rows_v.at[pl.ds(g * _IDXW, _IDXW), :],
                sem_g,
            )
            for g in range(_GPC)
        ]
        for h in handles:
            h.wait()
        pltpu.async_copy(rows_v, out_hbm.at[pl.ds(base, _CHUNK), :], sem_o)

    def pair_body(i, carry):
        do_chunk(2 * i, i, x_v0, idx_v0, rows_v0, sem_o0)
        do_chunk(2 * i + 1, i, x_v1, idx_v1, rows_v1, sem_o1)
        return carry

    lax.fori_loop(0, _NCHUNK // 2, pair_body, 0)
    pltpu.make_async_copy(rows_v0, out_hbm.at[pl.ds(0, _CHUNK), :], sem_o0).wait()
    pltpu.make_async_copy(rows_v1, out_hbm.at[pl.ds(0, _CHUNK), :], sem_o1).wait()


@jax.jit
def kernel(X, emb):
    X = X.astype(jnp.int32).reshape(-1)
    pe_flat = jnp.concatenate(
        [_pe_matrix().reshape(-1),
         jnp.zeros(_NS * _SPT * _D - _S * _D, dtype=jnp.float32)]
    )
    mesh = plsc.VectorSubcoreMesh(core_axis_name="c", subcore_axis_name="s")
    out = pl.kernel(
        _sc_body,
        mesh=mesh,
        compiler_params=pltpu.CompilerParams(use_tc_tiling_on_sc=False),
        out_type=jax.ShapeDtypeStruct((_ROWS, _D), jnp.float32),
        scratch_types=[
            pltpu.VMEM((_CHUNK,), jnp.int32),
            pltpu.VMEM((_CHUNK,), jnp.int32),
            pltpu.VMEM((_GPC, _IDXW), jnp.int32),
            pltpu.VMEM((_GPC, _IDXW), jnp.int32),
            pltpu.VMEM((_CHUNK, _D), jnp.float32),
            pltpu.VMEM((_CHUNK, _D), jnp.float32),
            pltpu.VMEM((_SPT * _D,), jnp.float32),
            pltpu.VMEM((_NN, _D), jnp.float32),
            pltpu.VMEM((_TROWS, _D), jnp.float32),
            pltpu.VMEM_SHARED((_TPAD, _D), jnp.float32),
            pltpu.SemaphoreType.DMA,
            pltpu.SemaphoreType.DMA,
            pltpu.SemaphoreType.DMA,
        ],
    )(X, pe_flat, emb)
    return out.reshape(_B, _S, _D)
